# Initial kernel scaffold; baseline (speedup 1.0000x reference)
import jax
import jax.numpy as jnp
from jax.experimental import pallas as pl


def kernel(x, batch, W1, b1, W2, b2, gamma, beta):
    raise NotImplementedError("write your pallas kernel here")


# speedup vs baseline: 8.6282x; 8.6282x over previous
import functools

import jax
import jax.numpy as jnp
from jax import lax
from jax.experimental import pallas as pl


def _segsum_body(batch_ref, x_ref, sums_ref, counts_ref, *, G):
    i = pl.program_id(0)

    @pl.when(i == 0)
    def _init():
        sums_ref[...] = jnp.zeros_like(sums_ref)
        counts_ref[...] = jnp.zeros_like(counts_ref)

    b = batch_ref[0, 0, :]
    BN = b.shape[0]
    ids = lax.broadcasted_iota(jnp.int32, (BN, G), 1)
    onehot = (b[:, None] == ids).astype(jnp.float32)
    x = x_ref[...]
    sums_ref[...] += lax.dot_general(
        onehot, x, (((0,), (0,)), ((), ())),
        preferred_element_type=jnp.float32)
    counts_ref[...] += jnp.sum(onehot, axis=0, keepdims=True)


def _ffn_body(sums_ref, counts_ref, W1_ref, b1_ref, W2_ref, b2_ref,
              gamma_ref, beta_ref, h_ref):
    counts = counts_ref[0, :]
    h = sums_ref[...] / jnp.clip(counts, 1.0)[:, None]
    h = jnp.maximum(
        lax.dot_general(h, W1_ref[...], (((1,), (0,)), ((), ())),
                        preferred_element_type=jnp.float32) + b1_ref[0, :],
        0.0)
    h = lax.dot_general(h, W2_ref[...], (((1,), (0,)), ((), ())),
                        preferred_element_type=jnp.float32) + b2_ref[0, :]
    mu = jnp.mean(h, axis=-1, keepdims=True)
    var = jnp.mean((h - mu) ** 2, axis=-1, keepdims=True)
    h = (h - mu) * lax.rsqrt(var + 1e-5) * gamma_ref[0, :] + beta_ref[0, :]
    h_ref[...] = h


def _bcast_body(batch_ref, x_ref, h_ref, out_ref, *, G):
    b = batch_ref[0, 0, :]
    BN = b.shape[0]
    ids = lax.broadcasted_iota(jnp.int32, (BN, G), 1)
    onehot = (b[:, None] == ids).astype(jnp.float32)
    out_ref[...] = x_ref[...] + lax.dot_general(
        onehot, h_ref[...], (((1,), (0,)), ((), ())),
        preferred_element_type=jnp.float32)


def kernel(x, batch, W1, b1, W2, b2, gamma, beta):
    N, D = x.shape
    G = 128
    BN = 2000
    NB = N // BN
    assert NB * BN == N
    batch3 = batch.astype(jnp.int32).reshape(NB, 1, BN)

    sums, counts = pl.pallas_call(
        functools.partial(_segsum_body, G=G),
        grid=(NB,),
        in_specs=[
            pl.BlockSpec((1, 1, BN), lambda i: (i, 0, 0)),
            pl.BlockSpec((BN, D), lambda i: (i, 0)),
        ],
        out_specs=[
            pl.BlockSpec((G, D), lambda i: (0, 0)),
            pl.BlockSpec((1, G), lambda i: (0, 0)),
        ],
        out_shape=[
            jax.ShapeDtypeStruct((G, D), jnp.float32),
            jax.ShapeDtypeStruct((1, G), jnp.float32),
        ],
    )(batch3, x)

    h_vn = pl.pallas_call(
        _ffn_body,
        in_specs=[
            pl.BlockSpec((G, D), lambda: (0, 0)),
            pl.BlockSpec((1, G), lambda: (0, 0)),
            pl.BlockSpec(W1.shape, lambda: (0, 0)),
            pl.BlockSpec((1, b1.shape[0]), lambda: (0, 0)),
            pl.BlockSpec(W2.shape, lambda: (0, 0)),
            pl.BlockSpec((1, b2.shape[0]), lambda: (0, 0)),
            pl.BlockSpec((1, D), lambda: (0, 0)),
            pl.BlockSpec((1, D), lambda: (0, 0)),
        ],
        out_specs=pl.BlockSpec((G, D), lambda: (0, 0)),
        out_shape=jax.ShapeDtypeStruct((G, D), jnp.float32),
    )(sums, counts, W1, b1.reshape(1, -1), W2, b2.reshape(1, -1),
      gamma.reshape(1, -1), beta.reshape(1, -1))

    x_out = pl.pallas_call(
        functools.partial(_bcast_body, G=G),
        grid=(NB,),
        in_specs=[
            pl.BlockSpec((1, 1, BN), lambda i: (i, 0, 0)),
            pl.BlockSpec((BN, D), lambda i: (i, 0)),
            pl.BlockSpec((G, D), lambda i: (0, 0)),
        ],
        out_specs=pl.BlockSpec((BN, D), lambda i: (i, 0)),
        out_shape=jax.ShapeDtypeStruct((N, D), jnp.float32),
    )(batch3, x, h_vn)

    return (x_out, h_vn)
